# baseline (speedup 1.0000x reference)
import functools

import jax
import jax.numpy as jnp
from jax import lax
from jax.experimental import pallas as pl
from jax.experimental.pallas import tpu as pltpu
from jax.experimental.pallas import tpu_sc as plsc

NC = 2
NS = 16
IW = 125
KD = 10
NB = 5
BLK = 2000


def _mesh():
    return plsc.VectorSubcoreMesh(
        core_axis_name="c", subcore_axis_name="s", num_cores=NC, num_subcores=NS
    )


@functools.lru_cache(maxsize=None)
def _deg_kernel(npad, nrows):
    rpt = npad // NS
    rpw = nrows // (NC * NS)
    nblk = rpw // KD

    @functools.partial(
        pl.kernel,
        out_type=jax.ShapeDtypeStruct((NC, npad, 16), jnp.float32),
        mesh=_mesh(),
        scratch_types=[
            pltpu.VMEM_SHARED((npad, 16), jnp.float32),
            pltpu.VMEM((2, KD, IW), jnp.int32),
            pltpu.VMEM((IW, 16), jnp.float32),
        ] + [pltpu.SemaphoreType.DMA] * (2 * KD + 2),
        compiler_params=pltpu.CompilerParams(use_tc_tiling_on_sc=False),
    )
    def deg(dst_hbm, zeros_hbm, ones_hbm, out_hbm, dacc, didxb, ones_v, *sems):
        ssem = (sems[:KD], sems[KD:2 * KD])
        isem = sems[2 * KD:]
        c = lax.axis_index("c")
        s = lax.axis_index("s")
        base = (s * NC + c) * rpw

        def stage(chunk, buf):
            pltpu.async_copy(dst_hbm.at[pl.ds(base + chunk * KD, KD)],
                             didxb.at[buf], isem[buf])

        def stage_wait(buf):
            pltpu.make_async_copy(dst_hbm.at[pl.ds(base, KD)],
                                  didxb.at[buf], isem[buf]).wait()

        pltpu.sync_copy(zeros_hbm, dacc.at[pl.ds(s * rpt, rpt)])
        pltpu.sync_copy(ones_hbm, ones_v)
        stage(0, 0)
        stage(1, 1)
        stage_wait(0)
        plsc.subcore_barrier()
        for r in range(KD):
            pltpu.async_copy(ones_v, dacc.at[didxb.at[0].at[r]], ssem[0][r],
                             add=True)

        def process_chunk(i, cur):
            nxt = 1 - cur

            @pl.when(i < nblk - 1)
            def _():
                stage_wait(nxt)
                for r in range(KD):
                    pltpu.async_copy(ones_v, dacc.at[didxb.at[nxt].at[r]],
                                     ssem[nxt][r], add=True)

            for r in range(KD):
                pltpu.make_async_copy(ones_v, dacc.at[didxb.at[cur].at[r]],
                                      ssem[cur][r]).wait()

            @pl.when(i < nblk - 2)
            def _():
                stage(i + 2, cur)

        def body(i2, carry):
            process_chunk(2 * i2, 0)
            process_chunk(2 * i2 + 1, 1)
            return carry

        lax.fori_loop(0, nblk // 2, body, 0)
        plsc.subcore_barrier()
        pltpu.sync_copy(dacc.at[pl.ds(s * rpt, rpt)],
                        out_hbm.at[c].at[pl.ds(s * rpt, rpt)])

    return deg


@functools.lru_cache(maxsize=None)
def _agg_kernel(npad, nrows, w):
    rpt = npad // NS
    rpw = nrows // NS
    nblk = rpw // NB

    @functools.partial(
        pl.kernel,
        out_type=jax.ShapeDtypeStruct((NC, npad, w), jnp.float32),
        mesh=_mesh(),
        scratch_types=[
            pltpu.VMEM_SHARED((npad, w), jnp.float32),
            pltpu.VMEM_SHARED((npad, w), jnp.float32),
            pltpu.VMEM((2, NB, IW), jnp.int32),
            pltpu.VMEM((2, NB, IW), jnp.int32),
            pltpu.VMEM((NB, IW, w), jnp.float32),
        ] + [pltpu.SemaphoreType.DMA] * (2 * NB + 2),
        compiler_params=pltpu.CompilerParams(use_tc_tiling_on_sc=False),
    )
    def agg(g_hbm, src_hbm, dst_hbm, out_hbm, acc, tbl, sidxb, didxb, rows,
            *sems):
        gsem = sems[:NB]
        ssem = sems[NB:2 * NB]
        isem = sems[2 * NB:]
        c = lax.axis_index("c")
        s = lax.axis_index("s")
        base = s * rpw

        def stage(chunk, buf, sem):
            pltpu.async_copy(src_hbm.at[pl.ds(base + chunk * NB, NB)],
                             sidxb.at[buf], sem)
            pltpu.async_copy(dst_hbm.at[pl.ds(base + chunk * NB, NB)],
                             didxb.at[buf], sem)

        def stage_wait(buf, sem):
            pltpu.make_async_copy(src_hbm.at[pl.ds(base, NB)],
                                  sidxb.at[buf], sem).wait()
            pltpu.make_async_copy(dst_hbm.at[pl.ds(base, NB)],
                                  didxb.at[buf], sem).wait()

        pltpu.sync_copy(g_hbm.at[c].at[pl.ds(s * rpt, rpt)],
                        tbl.at[pl.ds(s * rpt, rpt)])
        pltpu.sync_copy(g_hbm.at[c].at[pl.ds(s * rpt, rpt)],
                        acc.at[pl.ds(s * rpt, rpt)])
        stage(0, 0, isem[0])
        stage(1, 1, isem[1])
        stage_wait(0, isem[0])
        plsc.subcore_barrier()

        for r in range(NB):
            pltpu.async_copy(tbl.at[sidxb.at[0].at[r]], rows.at[r], gsem[r])

        def process_block(i, cur):
            nxt = 1 - cur
            for r in range(NB):
                pltpu.make_async_copy(tbl.at[sidxb.at[cur].at[r]], rows.at[r],
                                      gsem[r]).wait()
                pltpu.async_copy(rows.at[r], acc.at[didxb.at[cur].at[r]],
                                 ssem[r], add=True)

            @pl.when(i < nblk - 1)
            def _():
                stage_wait(nxt, isem[nxt])

            for r in range(NB):
                pltpu.make_async_copy(rows.at[r], acc.at[didxb.at[cur].at[r]],
                                      ssem[r]).wait()

                @pl.when(i < nblk - 1)
                def _():
                    pltpu.async_copy(tbl.at[sidxb.at[nxt].at[r]], rows.at[r],
                                     gsem[r])

            @pl.when(i < nblk - 2)
            def _():
                stage(i + 2, cur, isem[cur])

        def body(i2, carry):
            process_block(2 * i2, 0)
            process_block(2 * i2 + 1, 1)
            return carry

        lax.fori_loop(0, nblk // 2, body, 0)
        plsc.subcore_barrier()
        pltpu.sync_copy(acc.at[pl.ds(s * rpt, rpt)],
                        out_hbm.at[c].at[pl.ds(s * rpt, rpt)])

    return agg


def _tc_a0_body(x_ref, w1_ref, h_ref):
    h_ref[...] = jnp.dot(x_ref[...], w1_ref[...],
                         preferred_element_type=jnp.float32)


def _tc_a1_body(degp_ref, h_ref, g_ref, dinv_ref):
    deg = degp_ref[0, :, 0:1] + degp_ref[1, :, 0:1] + 1.0
    dinv = lax.rsqrt(deg)
    g = h_ref[...] * dinv
    g_ref[0] = g[:, :64]
    g_ref[1] = g[:, 64:]
    dinv_ref[...] = dinv


def _tc_b1_body(s_ref, dinv_ref, b1_ref, w2_ref, g_ref):
    dinv = dinv_ref[...]
    a0 = jnp.maximum(s_ref[0] * dinv + b1_ref[:, :64], 0.0)
    a1 = jnp.maximum(s_ref[1] * dinv + b1_ref[:, 64:], 0.0)
    h = (jnp.dot(a0, w2_ref[:64, :], preferred_element_type=jnp.float32)
         + jnp.dot(a1, w2_ref[64:, :], preferred_element_type=jnp.float32))
    g = h * dinv
    g_ref[0] = g[:, :64]
    g_ref[1] = g[:, 64:]


def _tc_b2_body(s_ref, dinv_ref, b2_ref, w3_ref, g_ref):
    dinv = dinv_ref[...]
    pre0 = s_ref[0] * dinv + b2_ref[:, :64]
    pre1 = s_ref[1] * dinv + b2_ref[:, 64:]
    m = jnp.maximum(jnp.max(pre0, axis=1, keepdims=True),
                    jnp.max(pre1, axis=1, keepdims=True))
    se = (jnp.sum(jnp.exp(pre0 - m), axis=1, keepdims=True)
          + jnp.sum(jnp.exp(pre1 - m), axis=1, keepdims=True))
    lse = m + jnp.log(se)
    a0 = pre0 - lse
    a1 = pre1 - lse
    h = (jnp.dot(a0, w3_ref[:64, :], preferred_element_type=jnp.float32)
         + jnp.dot(a1, w3_ref[64:, :], preferred_element_type=jnp.float32))
    g = h * dinv
    g_ref[0] = g[:, :32]
    g_ref[1] = g[:, 32:]


def _tc_c_body(nclass, s_ref, dinv_ref, b3_ref, out_ref):
    dinv = dinv_ref[...]
    pre0 = s_ref[0] * dinv + b3_ref[:, :32]
    pre1 = s_ref[1] * dinv + b3_ref[:, 32:]
    nreal = nclass - 32
    col = lax.broadcasted_iota(jnp.int32, pre1.shape, 1)
    msk = col < nreal
    pre1m = jnp.where(msk, pre1, -1e30)
    m = jnp.maximum(jnp.max(pre0, axis=1, keepdims=True),
                    jnp.max(pre1m, axis=1, keepdims=True))
    se = (jnp.sum(jnp.exp(pre0 - m), axis=1, keepdims=True)
          + jnp.sum(jnp.where(msk, jnp.exp(pre1 - m), 0.0), axis=1, keepdims=True))
    lse = m + jnp.log(se)
    out_ref[:, :32] = pre0 - lse
    out_ref[:, 32:] = (pre1 - lse)[:, :nreal]


def _grid_specs(n, npad):
    grid = (n // BLK,)
    half = pl.BlockSpec((NC, BLK, 64), lambda i: (0, i, 0))
    third = pl.BlockSpec((NC, BLK, 32), lambda i: (0, i, 0))
    dinv = pl.BlockSpec((BLK, 1), lambda i: (i, 0))
    full = lambda shape: pl.BlockSpec(shape, lambda i: tuple(0 for _ in shape))
    return grid, half, third, dinv, full


def kernel(x, edge_index, W1, b1, W2, b2, W3, b3):
    n, fin = x.shape
    e = edge_index.shape[1]
    hdim = W1.shape[1]
    nclass = W3.shape[1]
    assert fin == 128 and hdim == 128 and 32 < nclass <= 64
    assert e % (IW * NC * NS * KD) == 0 and n % BLK == 0

    npad = -(-n // 128) * 128
    nrows = e // IW

    src = edge_index[0].astype(jnp.int32).reshape(nrows, IW)
    dst = edge_index[1].astype(jnp.int32).reshape(nrows, IW)
    zeros16 = jnp.zeros((npad // NS, 16), jnp.float32)
    ones16 = jnp.ones((IW, 16), jnp.float32)
    W3p = jnp.pad(W3, ((0, 0), (0, 64 - nclass)))
    b1r = b1.reshape(1, 128)
    b2r = b2.reshape(1, 128)
    b3r = jnp.pad(b3, (0, 64 - nclass)).reshape(1, 64)

    degp = _deg_kernel(npad, nrows)(dst, zeros16, ones16)

    grid, half, third, dinv_s, full = _grid_specs(n, npad)
    degp_s = pl.BlockSpec((NC, BLK, 16), lambda i: (0, i, 0))
    xs = pl.BlockSpec((BLK, 128), lambda i: (i, 0))

    h1 = pl.pallas_call(
        _tc_a0_body,
        grid=grid,
        in_specs=[xs, full((128, 128))],
        out_specs=xs,
        out_shape=jax.ShapeDtypeStruct((n, 128), jnp.float32),
    )(x, W1)

    g1, dinv = pl.pallas_call(
        _tc_a1_body,
        grid=grid,
        in_specs=[degp_s, xs],
        out_specs=[half, dinv_s],
        out_shape=[
            jax.ShapeDtypeStruct((NC, npad, 64), jnp.float32),
            jax.ShapeDtypeStruct((n, 1), jnp.float32),
        ],
    )(degp, h1)

    agg64 = _agg_kernel(npad, nrows, 64)
    agg32 = _agg_kernel(npad, nrows, 32)

    s1 = agg64(g1, src, dst)

    g2 = pl.pallas_call(
        _tc_b1_body,
        grid=grid,
        in_specs=[half, dinv_s, full((1, 128)), full((128, 128))],
        out_specs=half,
        out_shape=jax.ShapeDtypeStruct((NC, npad, 64), jnp.float32),
    )(s1, dinv, b1r, W2)

    s2 = agg64(g2, src, dst)

    g3 = pl.pallas_call(
        _tc_b2_body,
        grid=grid,
        in_specs=[half, dinv_s, full((1, 128)), full((128, 64))],
        out_specs=third,
        out_shape=jax.ShapeDtypeStruct((NC, npad, 32), jnp.float32),
    )(s2, dinv, b2r, W3p)

    s3 = agg32(g3, src, dst)

    out = pl.pallas_call(
        functools.partial(_tc_c_body, nclass),
        grid=grid,
        in_specs=[third, dinv_s, full((1, 64))],
        out_specs=pl.BlockSpec((BLK, nclass), lambda i: (i, 0)),
        out_shape=jax.ShapeDtypeStruct((n, nclass), jnp.float32),
    )(s3, dinv, b3r)

    return out


# speedup vs baseline: 1.4007x; 1.4007x over previous
import functools

import jax
import jax.numpy as jnp
from jax import lax
from jax.experimental import pallas as pl
from jax.experimental.pallas import tpu as pltpu
from jax.experimental.pallas import tpu_sc as plsc

NC = 2
NS = 16
IW = 125
KD = 10
NB = 10
BLK = 2000


def _mesh():
    return plsc.VectorSubcoreMesh(
        core_axis_name="c", subcore_axis_name="s", num_cores=NC, num_subcores=NS
    )


@functools.lru_cache(maxsize=None)
def _deg_kernel(npad, nrows):
    rpt = npad // NS
    rpw = nrows // (NC * NS)
    nblk = rpw // KD

    @functools.partial(
        pl.kernel,
        out_type=jax.ShapeDtypeStruct((NC, npad, 16), jnp.float32),
        mesh=_mesh(),
        scratch_types=[
            pltpu.VMEM_SHARED((npad, 16), jnp.float32),
            pltpu.VMEM((2, KD, IW), jnp.int32),
            pltpu.VMEM((IW, 16), jnp.float32),
        ] + [pltpu.SemaphoreType.DMA] * (2 * KD + 2),
        compiler_params=pltpu.CompilerParams(use_tc_tiling_on_sc=False),
    )
    def deg(dst_hbm, zeros_hbm, ones_hbm, out_hbm, dacc, didxb, ones_v, *sems):
        ssem = (sems[:KD], sems[KD:2 * KD])
        isem = sems[2 * KD:]
        c = lax.axis_index("c")
        s = lax.axis_index("s")
        base = (s * NC + c) * rpw

        def stage(chunk, buf):
            pltpu.async_copy(dst_hbm.at[pl.ds(base + chunk * KD, KD)],
                             didxb.at[buf], isem[buf])

        def stage_wait(buf):
            pltpu.make_async_copy(dst_hbm.at[pl.ds(base, KD)],
                                  didxb.at[buf], isem[buf]).wait()

        pltpu.sync_copy(zeros_hbm, dacc.at[pl.ds(s * rpt, rpt)])
        pltpu.sync_copy(ones_hbm, ones_v)
        stage(0, 0)
        stage(1, 1)
        stage_wait(0)
        plsc.subcore_barrier()
        for r in range(KD):
            pltpu.async_copy(ones_v, dacc.at[didxb.at[0].at[r]], ssem[0][r],
                             add=True)

        def process_chunk(i, cur):
            nxt = 1 - cur

            @pl.when(i < nblk - 1)
            def _():
                stage_wait(nxt)
                for r in range(KD):
                    pltpu.async_copy(ones_v, dacc.at[didxb.at[nxt].at[r]],
                                     ssem[nxt][r], add=True)

            for r in range(KD):
                pltpu.make_async_copy(ones_v, dacc.at[didxb.at[cur].at[r]],
                                      ssem[cur][r]).wait()

            @pl.when(i < nblk - 2)
            def _():
                stage(i + 2, cur)

        def body(i2, carry):
            process_chunk(2 * i2, 0)
            process_chunk(2 * i2 + 1, 1)
            return carry

        lax.fori_loop(0, nblk // 2, body, 0)
        plsc.subcore_barrier()
        pltpu.sync_copy(dacc.at[pl.ds(s * rpt, rpt)],
                        out_hbm.at[c].at[pl.ds(s * rpt, rpt)])

    return deg


@functools.lru_cache(maxsize=None)
def _agg_kernel(npad, nrows, w):
    rpt = npad // NS
    rpw = nrows // NS
    nblk = rpw // NB

    @functools.partial(
        pl.kernel,
        out_type=jax.ShapeDtypeStruct((NC, npad, w), jnp.float32),
        mesh=_mesh(),
        scratch_types=[
            pltpu.VMEM_SHARED((npad, w), jnp.float32),
            pltpu.VMEM((2, NB, IW), jnp.int32),
            pltpu.VMEM((2, NB, IW), jnp.int32),
            pltpu.VMEM((NB, IW, w), jnp.float32),
        ] + [pltpu.SemaphoreType.DMA] * (2 * NB + 2),
        compiler_params=pltpu.CompilerParams(use_tc_tiling_on_sc=False),
    )
    def agg(g_hbm, src_hbm, dst_hbm, out_hbm, acc, sidxb, didxb, rows, *sems):
        gsem = sems[:NB]
        ssem = sems[NB:2 * NB]
        isem = sems[2 * NB:]
        c = lax.axis_index("c")
        s = lax.axis_index("s")
        tbl = g_hbm.at[c]
        base = s * rpw

        def stage(chunk, buf, sem):
            pltpu.async_copy(src_hbm.at[pl.ds(base + chunk * NB, NB)],
                             sidxb.at[buf], sem)
            pltpu.async_copy(dst_hbm.at[pl.ds(base + chunk * NB, NB)],
                             didxb.at[buf], sem)

        def stage_wait(buf, sem):
            pltpu.make_async_copy(src_hbm.at[pl.ds(base, NB)],
                                  sidxb.at[buf], sem).wait()
            pltpu.make_async_copy(dst_hbm.at[pl.ds(base, NB)],
                                  didxb.at[buf], sem).wait()

        pltpu.sync_copy(tbl.at[pl.ds(s * rpt, rpt)], acc.at[pl.ds(s * rpt, rpt)])
        stage(0, 0, isem[0])
        stage(1, 1, isem[1])
        stage_wait(0, isem[0])
        plsc.subcore_barrier()

        for r in range(NB):
            pltpu.async_copy(tbl.at[sidxb.at[0].at[r]], rows.at[r], gsem[r])

        def process_block(i, cur):
            nxt = 1 - cur
            for r in range(NB):
                pltpu.make_async_copy(tbl.at[sidxb.at[cur].at[r]], rows.at[r],
                                      gsem[r]).wait()
                pltpu.async_copy(rows.at[r], acc.at[didxb.at[cur].at[r]],
                                 ssem[r], add=True)

            @pl.when(i < nblk - 1)
            def _():
                stage_wait(nxt, isem[nxt])

            for r in range(NB):
                pltpu.make_async_copy(rows.at[r], acc.at[didxb.at[cur].at[r]],
                                      ssem[r]).wait()

                @pl.when(i < nblk - 1)
                def _():
                    pltpu.async_copy(tbl.at[sidxb.at[nxt].at[r]], rows.at[r],
                                     gsem[r])

            @pl.when(i < nblk - 2)
            def _():
                stage(i + 2, cur, isem[cur])

        def body(i2, carry):
            process_block(2 * i2, 0)
            process_block(2 * i2 + 1, 1)
            return carry

        lax.fori_loop(0, nblk // 2, body, 0)
        plsc.subcore_barrier()
        pltpu.sync_copy(acc.at[pl.ds(s * rpt, rpt)],
                        out_hbm.at[c].at[pl.ds(s * rpt, rpt)])

    return agg


def _tc_a0_body(x_ref, w1_ref, h_ref):
    h_ref[...] = jnp.dot(x_ref[...], w1_ref[...],
                         preferred_element_type=jnp.float32)


def _tc_a1_body(degp_ref, h_ref, g_ref, dinv_ref):
    deg = degp_ref[0, :, 0:1] + degp_ref[1, :, 0:1] + 1.0
    dinv = lax.rsqrt(deg)
    g = h_ref[...] * dinv
    g_ref[0] = g[:, :64]
    g_ref[1] = g[:, 64:]
    dinv_ref[...] = dinv


def _tc_b1_body(s_ref, dinv_ref, b1_ref, w2_ref, g_ref):
    dinv = dinv_ref[...]
    a0 = jnp.maximum(s_ref[0] * dinv + b1_ref[:, :64], 0.0)
    a1 = jnp.maximum(s_ref[1] * dinv + b1_ref[:, 64:], 0.0)
    h = (jnp.dot(a0, w2_ref[:64, :], preferred_element_type=jnp.float32)
         + jnp.dot(a1, w2_ref[64:, :], preferred_element_type=jnp.float32))
    g = h * dinv
    g_ref[0] = g[:, :64]
    g_ref[1] = g[:, 64:]


def _tc_b2_body(s_ref, dinv_ref, b2_ref, w3_ref, g_ref):
    dinv = dinv_ref[...]
    pre0 = s_ref[0] * dinv + b2_ref[:, :64]
    pre1 = s_ref[1] * dinv + b2_ref[:, 64:]
    m = jnp.maximum(jnp.max(pre0, axis=1, keepdims=True),
                    jnp.max(pre1, axis=1, keepdims=True))
    se = (jnp.sum(jnp.exp(pre0 - m), axis=1, keepdims=True)
          + jnp.sum(jnp.exp(pre1 - m), axis=1, keepdims=True))
    lse = m + jnp.log(se)
    a0 = pre0 - lse
    a1 = pre1 - lse
    h = (jnp.dot(a0, w3_ref[:64, :], preferred_element_type=jnp.float32)
         + jnp.dot(a1, w3_ref[64:, :], preferred_element_type=jnp.float32))
    g = h * dinv
    g_ref[0] = g[:, :32]
    g_ref[1] = g[:, 32:]


def _tc_c_body(nclass, s_ref, dinv_ref, b3_ref, out_ref):
    dinv = dinv_ref[...]
    pre0 = s_ref[0] * dinv + b3_ref[:, :32]
    pre1 = s_ref[1] * dinv + b3_ref[:, 32:]
    nreal = nclass - 32
    col = lax.broadcasted_iota(jnp.int32, pre1.shape, 1)
    msk = col < nreal
    pre1m = jnp.where(msk, pre1, -1e30)
    m = jnp.maximum(jnp.max(pre0, axis=1, keepdims=True),
                    jnp.max(pre1m, axis=1, keepdims=True))
    se = (jnp.sum(jnp.exp(pre0 - m), axis=1, keepdims=True)
          + jnp.sum(jnp.where(msk, jnp.exp(pre1 - m), 0.0), axis=1, keepdims=True))
    lse = m + jnp.log(se)
    out_ref[:, :32] = pre0 - lse
    out_ref[:, 32:] = (pre1 - lse)[:, :nreal]


def _grid_specs(n, npad):
    grid = (n // BLK,)
    half = pl.BlockSpec((NC, BLK, 64), lambda i: (0, i, 0))
    third = pl.BlockSpec((NC, BLK, 32), lambda i: (0, i, 0))
    dinv = pl.BlockSpec((BLK, 1), lambda i: (i, 0))
    full = lambda shape: pl.BlockSpec(shape, lambda i: tuple(0 for _ in shape))
    return grid, half, third, dinv, full


def kernel(x, edge_index, W1, b1, W2, b2, W3, b3):
    n, fin = x.shape
    e = edge_index.shape[1]
    hdim = W1.shape[1]
    nclass = W3.shape[1]
    assert fin == 128 and hdim == 128 and 32 < nclass <= 64
    assert e % (IW * NC * NS * KD) == 0 and n % BLK == 0

    npad = -(-n // 128) * 128
    nrows = e // IW

    src = edge_index[0].astype(jnp.int32).reshape(nrows, IW)
    dst = edge_index[1].astype(jnp.int32).reshape(nrows, IW)
    zeros16 = jnp.zeros((npad // NS, 16), jnp.float32)
    ones16 = jnp.ones((IW, 16), jnp.float32)
    W3p = jnp.pad(W3, ((0, 0), (0, 64 - nclass)))
    b1r = b1.reshape(1, 128)
    b2r = b2.reshape(1, 128)
    b3r = jnp.pad(b3, (0, 64 - nclass)).reshape(1, 64)

    degp = _deg_kernel(npad, nrows)(dst, zeros16, ones16)

    grid, half, third, dinv_s, full = _grid_specs(n, npad)
    degp_s = pl.BlockSpec((NC, BLK, 16), lambda i: (0, i, 0))
    xs = pl.BlockSpec((BLK, 128), lambda i: (i, 0))

    h1 = pl.pallas_call(
        _tc_a0_body,
        grid=grid,
        in_specs=[xs, full((128, 128))],
        out_specs=xs,
        out_shape=jax.ShapeDtypeStruct((n, 128), jnp.float32),
    )(x, W1)

    g1, dinv = pl.pallas_call(
        _tc_a1_body,
        grid=grid,
        in_specs=[degp_s, xs],
        out_specs=[half, dinv_s],
        out_shape=[
            jax.ShapeDtypeStruct((NC, npad, 64), jnp.float32),
            jax.ShapeDtypeStruct((n, 1), jnp.float32),
        ],
    )(degp, h1)

    agg64 = _agg_kernel(npad, nrows, 64)
    agg32 = _agg_kernel(npad, nrows, 32)

    s1 = agg64(g1, src, dst)

    g2 = pl.pallas_call(
        _tc_b1_body,
        grid=grid,
        in_specs=[half, dinv_s, full((1, 128)), full((128, 128))],
        out_specs=half,
        out_shape=jax.ShapeDtypeStruct((NC, npad, 64), jnp.float32),
    )(s1, dinv, b1r, W2)

    s2 = agg64(g2, src, dst)

    g3 = pl.pallas_call(
        _tc_b2_body,
        grid=grid,
        in_specs=[half, dinv_s, full((1, 128)), full((128, 64))],
        out_specs=third,
        out_shape=jax.ShapeDtypeStruct((NC, npad, 32), jnp.float32),
    )(s2, dinv, b2r, W3p)

    s3 = agg32(g3, src, dst)

    out = pl.pallas_call(
        functools.partial(_tc_c_body, nclass),
        grid=grid,
        in_specs=[third, dinv_s, full((1, 64))],
        out_specs=pl.BlockSpec((BLK, nclass), lambda i: (i, 0)),
        out_shape=jax.ShapeDtypeStruct((n, nclass), jnp.float32),
    )(s3, dinv, b3r)

    return out
